# baseline (speedup 1.0000x reference)
import jax
import jax.numpy as jnp
from jax import lax
from jax.experimental import pallas as pl
from jax.experimental.pallas import tpu as pltpu

_DIM = 1024
_BATCH = 32
_DBLK = 32
_NSTEPS = _DIM // _DBLK
_NBUF = 4


def _tc_body(t_ref, table_ref, pos_ref, out_ref, rows_ref, slabs_ref, gsem, osem):
    i = pl.program_id(0)

    @pl.when(i == 0)
    def _gather():
        copies = []
        for b in range(_BATCH):
            cp = pltpu.make_async_copy(
                table_ref.at[pl.ds(t_ref[b], 1), :],
                rows_ref.at[pl.ds(b, 1), :],
                gsem,
            )
            cp.start()
            copies.append(cp)
        for cp in copies:
            cp.wait()

    pos_vals = pos_ref[0, :, 0, 0]
    slab = pos_vals[:, None, None] + rows_ref[:, :][None, :, :]

    buf = lax.rem(i, _NBUF)
    for j in range(_NBUF):

        @pl.when(buf == j)
        def _run(j=j):
            dma = pltpu.make_async_copy(
                slabs_ref.at[j],
                out_ref.at[0, pl.ds(i * _DBLK, _DBLK), :, :],
                osem.at[j],
            )
            @pl.when(i >= _NBUF)
            def _drain():
                dma.wait()

            slabs_ref[j] = slab
            dma.start()

    @pl.when(i == _NSTEPS - 1)
    def _tail():
        for j in range(_NBUF):
            pltpu.make_async_copy(
                slabs_ref.at[j],
                out_ref.at[0, pl.ds(0, _DBLK), :, :],
                osem.at[j],
            ).wait()


def kernel(t, table, pos_emb):
    t = t.astype(jnp.int32)
    return pl.pallas_call(
        _tc_body,
        grid_spec=pltpu.PrefetchScalarGridSpec(
            num_scalar_prefetch=1,
            grid=(_NSTEPS,),
            in_specs=[
                pl.BlockSpec(memory_space=pltpu.MemorySpace.HBM),
                pl.BlockSpec((1, _DBLK, 1, 1), lambda i, t_pref: (0, i, 0, 0)),
            ],
            out_specs=pl.BlockSpec(memory_space=pltpu.MemorySpace.HBM),
            scratch_shapes=[
                pltpu.VMEM((_BATCH, _DIM), jnp.float32),
                pltpu.VMEM((_NBUF, _DBLK, _BATCH, _DIM), jnp.float32),
                pltpu.SemaphoreType.DMA,
                pltpu.SemaphoreType.DMA((_NBUF,)),
            ],
        ),
        out_shape=jax.ShapeDtypeStruct((1, _DIM, _BATCH, _DIM), jnp.float32),
    )(t, table, pos_emb)


# speedup vs baseline: 1.0348x; 1.0348x over previous
import jax
import jax.numpy as jnp
from jax.experimental import pallas as pl
from jax.experimental.pallas import tpu as pltpu

_DIM = 1024
_BATCH = 32
_DBLK = 32


def _tc_body(t_ref, table_ref, pos_ref, out_ref, rows_ref, sem):
    i = pl.program_id(0)

    @pl.when(i == 0)
    def _gather():
        copies = []
        for b in range(_BATCH):
            cp = pltpu.make_async_copy(
                table_ref.at[pl.ds(t_ref[b], 1), :],
                rows_ref.at[pl.ds(b, 1), :],
                sem,
            )
            cp.start()
            copies.append(cp)
        for cp in copies:
            cp.wait()

    pos_vals = pos_ref[0, pl.ds(i * _DBLK, _DBLK), 0, 0]
    out_ref[0] = pos_vals[:, None, None] + rows_ref[:, :][None, :, :]


def kernel(t, table, pos_emb):
    t = t.astype(jnp.int32)
    grid = (_DIM // _DBLK,)
    return pl.pallas_call(
        _tc_body,
        grid_spec=pltpu.PrefetchScalarGridSpec(
            num_scalar_prefetch=1,
            grid=grid,
            in_specs=[
                pl.BlockSpec(memory_space=pltpu.MemorySpace.HBM),
                pl.BlockSpec((1, _DIM, 1, 1), lambda i, t_pref: (0, 0, 0, 0)),
            ],
            out_specs=pl.BlockSpec(
                (1, _DBLK, _BATCH, _DIM), lambda i, t_pref: (0, i, 0, 0)
            ),
            scratch_shapes=[
                pltpu.VMEM((_BATCH, _DIM), jnp.float32),
                pltpu.SemaphoreType.DMA,
            ],
        ),
        out_shape=jax.ShapeDtypeStruct((1, _DIM, _BATCH, _DIM), jnp.float32),
    )(t, table, pos_emb)
